# baseline (speedup 1.0000x reference)
import functools

import jax
import jax.numpy as jnp
from jax.experimental import pallas as pl
from jax.experimental.pallas import tpu as pltpu

_K = 128

_BT1 = 512
_BH = 4096
_BH2 = 1024
_BT2 = 128
_BT3 = 1024


def _mm1_kernel(x_ref, w1_ref, b1_ref, h_ref):
    acc = jax.lax.dot_general(
        x_ref[...], w1_ref[...], (((1,), (1,)), ((), ())),
        preferred_element_type=jnp.float32)
    h_ref[...] = acc + b1_ref[...]


def _thresh_kernel(h_ref, t_ref, *, kk):
    h = h_ref[...]
    n = h.shape[1]
    rows = h.shape[0]

    vlo = jnp.min(h, axis=1, keepdims=True)
    vhi = jnp.max(h, axis=1, keepdims=True)
    clo = jnp.full((rows, 1), float(n), dtype=jnp.float32)
    chi = jnp.ones((rows, 1), dtype=jnp.float32)

    def zof(c):
        return jnp.sqrt(2.0 * jnp.log(float(n) / jnp.maximum(c, 0.5)))

    def to_key(v):
        b = jax.lax.bitcast_convert_type(v, jnp.int32)
        return jnp.where(b < 0, jnp.bitwise_xor(b, jnp.int32(0x7FFFFFFF)), b)

    def from_key(k):
        b = jnp.where(k < 0, jnp.bitwise_xor(k, jnp.int32(0x7FFFFFFF)), k)
        return jax.lax.bitcast_convert_type(b, jnp.float32)

    _INTERP = 16
    _MAX = _INTERP + 32

    def cond(state):
        i, vlo, vhi, clo, chi = state
        unconv = jnp.sum(jnp.where(clo == kk, 0.0, 1.0))
        return jnp.logical_and(i < _MAX, unconv > 0.0)

    def body(state):
        i, vlo, vhi, clo, chi = state
        zlo, zhi = zof(clo), zof(chi)
        frac = (zhi - zof(jnp.float32(kk))) / jnp.maximum(zhi - zlo, 1e-9)
        cand_i = vhi + (vlo - vhi) * frac
        cand_i = jnp.clip(cand_i, vlo, vhi)
        klo, khi = to_key(vlo), to_key(vhi)
        cand_b = from_key((klo & khi) + ((klo ^ khi) >> 1))
        cand = jnp.where(i < _INTERP, cand_i, cand_b)
        cnt = jnp.sum((h >= cand).astype(jnp.float32), axis=1, keepdims=True)
        ge = cnt >= kk
        vlo = jnp.where(ge, cand, vlo)
        clo = jnp.where(ge, cnt, clo)
        vhi = jnp.where(ge, vhi, cand)
        chi = jnp.where(ge, chi, cnt)
        return i + jnp.int32(1), vlo, vhi, clo, chi

    _, vlo, _, _, _ = jax.lax.while_loop(
        cond, body, (jnp.int32(0), vlo, vhi, clo, chi))
    t_ref[...] = vlo


def _mm2_kernel(h_ref, t_ref, w2_ref, b2_ref, y_ref):
    k = pl.program_id(1)
    h = h_ref[...]
    sp = jnp.where(h >= t_ref[...], h, 0.0).astype(jnp.bfloat16)
    acc = jax.lax.dot_general(
        sp, w2_ref[...], (((1,), (1,)), ((), ())),
        preferred_element_type=jnp.float32)

    @pl.when(k == 0)
    def _():
        y_ref[...] = acc + b2_ref[...]

    @pl.when(k != 0)
    def _():
        y_ref[...] += acc


def kernel(x, W1, b1, W2, b2):
    T, D = x.shape
    H = W1.shape[0]
    O = W2.shape[0]
    bt1, bh, bt2, bt3 = (min(_BT1, T), min(_BH, H), min(_BT2, T), min(_BT3, T))
    bh2 = min(_BH2, H)

    xb = x.astype(jnp.bfloat16)
    w1b = W1.astype(jnp.bfloat16)
    w2b = W2.astype(jnp.bfloat16)
    b1r = b1.reshape(1, H)
    b2r = b2.reshape(1, O)

    h = pl.pallas_call(
        _mm1_kernel,
        grid=(H // bh, T // bt1),
        in_specs=[
            pl.BlockSpec((bt1, D), lambda j, i: (i, 0)),
            pl.BlockSpec((bh, D), lambda j, i: (j, 0)),
            pl.BlockSpec((1, bh), lambda j, i: (0, j)),
        ],
        out_specs=pl.BlockSpec((bt1, bh), lambda j, i: (i, j)),
        out_shape=jax.ShapeDtypeStruct((T, H), jnp.float32),
        compiler_params=pltpu.CompilerParams(
            dimension_semantics=("parallel", "parallel")),
    )(xb, w1b, b1r)

    t = pl.pallas_call(
        functools.partial(_thresh_kernel, kk=_K),
        grid=(T // bt2,),
        in_specs=[pl.BlockSpec((bt2, H), lambda i: (i, 0))],
        out_specs=pl.BlockSpec((bt2, 1), lambda i: (i, 0)),
        out_shape=jax.ShapeDtypeStruct((T, 1), jnp.float32),
        compiler_params=pltpu.CompilerParams(
            dimension_semantics=("parallel",)),
    )(h)

    y = pl.pallas_call(
        _mm2_kernel,
        grid=(T // bt3, H // bh2),
        in_specs=[
            pl.BlockSpec((bt3, bh2), lambda i, k: (i, k)),
            pl.BlockSpec((bt3, 1), lambda i, k: (i, 0)),
            pl.BlockSpec((O, bh2), lambda i, k: (0, k)),
            pl.BlockSpec((1, O), lambda i, k: (0, 0)),
        ],
        out_specs=pl.BlockSpec((bt3, O), lambda i, k: (i, 0)),
        out_shape=jax.ShapeDtypeStruct((T, O), jnp.float32),
        compiler_params=pltpu.CompilerParams(
            dimension_semantics=("parallel", "arbitrary")),
    )(h, t, w2b, b2r)

    return y


# speedup vs baseline: 1.0129x; 1.0129x over previous
import functools

import jax
import jax.numpy as jnp
from jax.experimental import pallas as pl
from jax.experimental.pallas import tpu as pltpu

_K = 128

_BT1 = 512
_BH = 4096
_BH2 = 1024
_BT2 = 256
_BT3 = 1024


def _mm1_kernel(x_ref, w1_ref, b1_ref, h_ref):
    acc = jax.lax.dot_general(
        x_ref[...], w1_ref[...], (((1,), (1,)), ((), ())),
        preferred_element_type=jnp.float32)
    h_ref[...] = acc + b1_ref[...]


def _thresh_kernel(h_ref, t_ref, *, kk):
    h = h_ref[...]
    n = h.shape[1]
    rows = h.shape[0]

    vlo = jnp.min(h, axis=1, keepdims=True)
    vhi = jnp.max(h, axis=1, keepdims=True)
    clo = jnp.full((rows, 1), float(n), dtype=jnp.float32)
    chi = jnp.ones((rows, 1), dtype=jnp.float32)

    def zof(c):
        return jnp.sqrt(2.0 * jnp.log(float(n) / jnp.maximum(c, 0.5)))

    def to_key(v):
        b = jax.lax.bitcast_convert_type(v, jnp.int32)
        return jnp.where(b < 0, jnp.bitwise_xor(b, jnp.int32(0x7FFFFFFF)), b)

    def from_key(k):
        b = jnp.where(k < 0, jnp.bitwise_xor(k, jnp.int32(0x7FFFFFFF)), k)
        return jax.lax.bitcast_convert_type(b, jnp.float32)

    _INTERP = 16
    _MAX = _INTERP + 32

    def cond(state):
        i, vlo, vhi, clo, chi = state
        unconv = jnp.sum(jnp.where(clo == kk, 0.0, 1.0))
        return jnp.logical_and(i < _MAX, unconv > 0.0)

    def body(state):
        i, vlo, vhi, clo, chi = state
        zlo, zhi = zof(clo), zof(chi)
        frac = (zhi - zof(jnp.float32(kk))) / jnp.maximum(zhi - zlo, 1e-9)
        cand_i = vhi + (vlo - vhi) * frac
        cand_i = jnp.clip(cand_i, vlo, vhi)
        klo, khi = to_key(vlo), to_key(vhi)
        cand_b = from_key((klo & khi) + ((klo ^ khi) >> 1))
        cand = jnp.where(i < _INTERP, cand_i, cand_b)
        cnt = jnp.sum((h >= cand).astype(jnp.float32), axis=1, keepdims=True)
        ge = cnt >= kk
        vlo = jnp.where(ge, cand, vlo)
        clo = jnp.where(ge, cnt, clo)
        vhi = jnp.where(ge, vhi, cand)
        chi = jnp.where(ge, chi, cnt)
        return i + jnp.int32(1), vlo, vhi, clo, chi

    _, vlo, _, _, _ = jax.lax.while_loop(
        cond, body, (jnp.int32(0), vlo, vhi, clo, chi))
    t_ref[...] = vlo


def _mm2_kernel(h_ref, t_ref, w2_ref, b2_ref, y_ref):
    k = pl.program_id(1)
    h = h_ref[...]
    sp = jnp.where(h >= t_ref[...], h, 0.0).astype(jnp.bfloat16)
    acc = jax.lax.dot_general(
        sp, w2_ref[...], (((1,), (1,)), ((), ())),
        preferred_element_type=jnp.float32)

    @pl.when(k == 0)
    def _():
        y_ref[...] = acc + b2_ref[...]

    @pl.when(k != 0)
    def _():
        y_ref[...] += acc


def kernel(x, W1, b1, W2, b2):
    T, D = x.shape
    H = W1.shape[0]
    O = W2.shape[0]
    bt1, bh, bt2, bt3 = (min(_BT1, T), min(_BH, H), min(_BT2, T), min(_BT3, T))
    bh2 = min(_BH2, H)

    xb = x.astype(jnp.bfloat16)
    w1b = W1.astype(jnp.bfloat16)
    w2b = W2.astype(jnp.bfloat16)
    b1r = b1.reshape(1, H)
    b2r = b2.reshape(1, O)

    h = pl.pallas_call(
        _mm1_kernel,
        grid=(H // bh, T // bt1),
        in_specs=[
            pl.BlockSpec((bt1, D), lambda j, i: (i, 0)),
            pl.BlockSpec((bh, D), lambda j, i: (j, 0)),
            pl.BlockSpec((1, bh), lambda j, i: (0, j)),
        ],
        out_specs=pl.BlockSpec((bt1, bh), lambda j, i: (i, j)),
        out_shape=jax.ShapeDtypeStruct((T, H), jnp.float32),
        compiler_params=pltpu.CompilerParams(
            dimension_semantics=("parallel", "parallel")),
    )(xb, w1b, b1r)

    t = pl.pallas_call(
        functools.partial(_thresh_kernel, kk=_K),
        grid=(T // bt2,),
        in_specs=[pl.BlockSpec((bt2, H), lambda i: (i, 0))],
        out_specs=pl.BlockSpec((bt2, 1), lambda i: (i, 0)),
        out_shape=jax.ShapeDtypeStruct((T, 1), jnp.float32),
        compiler_params=pltpu.CompilerParams(
            dimension_semantics=("parallel",)),
    )(h)

    y = pl.pallas_call(
        _mm2_kernel,
        grid=(T // bt3, H // bh2),
        in_specs=[
            pl.BlockSpec((bt3, bh2), lambda i, k: (i, k)),
            pl.BlockSpec((bt3, 1), lambda i, k: (i, 0)),
            pl.BlockSpec((O, bh2), lambda i, k: (0, k)),
            pl.BlockSpec((1, O), lambda i, k: (0, 0)),
        ],
        out_specs=pl.BlockSpec((bt3, O), lambda i, k: (i, 0)),
        out_shape=jax.ShapeDtypeStruct((T, O), jnp.float32),
        compiler_params=pltpu.CompilerParams(
            dimension_semantics=("parallel", "arbitrary")),
    )(h, t, w2b, b2r)

    return y
